# baseline (speedup 1.0000x reference)
import functools

import jax
import jax.numpy as jnp
from jax import lax
from jax.experimental import pallas as pl
from jax.experimental.pallas import tpu as pltpu
from jax.experimental.pallas import tpu_sc as plsc

N = 1_000_000
NP = 1_003_520
NW = 32
NC = 2
CHUNK = NP // NW
NB = 1024
SB = 6272
NSB = CHUNK // SB
ROWS = SB // 128
PAD_CODE = (1 << 30) - 1


C = 6272
G = NP // C


def _expand3(v):
    x = v
    x = (x | (x << 16)) & 0x30000FF
    x = (x | (x << 8)) & 0x300F00F
    x = (x | (x << 4)) & 0x30C30C3
    x = (x | (x << 2)) & 0x9249249
    return x


def _morton_body(pc_ref, codes_ref, mm_ref):
    ph = pl.program_id(0)
    g = pl.program_id(1)

    @pl.when(ph == 0)
    def _():
        blk = pc_ref[...]
        bmin = jnp.min(blk, axis=1, keepdims=True)
        bmax = jnp.max(blk, axis=1, keepdims=True)
        prev_min = jnp.where(g == 0, jnp.full_like(bmin, jnp.inf), mm_ref[:, 0:1])
        prev_max = jnp.where(g == 0, jnp.full_like(bmax, -jnp.inf), mm_ref[:, 1:2])
        mm_ref[:, 0:1] = jnp.minimum(prev_min, bmin)
        mm_ref[:, 1:2] = jnp.maximum(prev_max, bmax)

    @pl.when(ph == 1)
    def _():
        blk = pc_ref[...]
        bmin = mm_ref[:, 0:1]
        bmax = mm_ref[:, 1:2]
        scale = jnp.float32(1023) / (bmax - bmin + jnp.float32(1e-7))
        q = jnp.floor((blk - bmin) * scale).astype(jnp.int32)
        q = jnp.minimum(q, 1023)
        e = _expand3(q)
        code = (e[0:1, :] << 2) | (e[1:2, :] << 1) | e[2:3, :]
        pos = g * C + lax.broadcasted_iota(jnp.int32, (1, C), 1)
        codes_ref[...] = jnp.where(pos < N, code, PAD_CODE)


def _morton_codes_padded(pcp):
    return pl.pallas_call(
        _morton_body,
        grid=(2, G),
        in_specs=[pl.BlockSpec((3, C), lambda ph, g: (0, g))],
        out_specs=pl.BlockSpec((1, C), lambda ph, g: (0, g)),
        out_shape=jax.ShapeDtypeStruct((1, NP), jnp.int32),
        scratch_shapes=[pltpu.VMEM((3, 2), jnp.float32)],
    )(pcp)


_SC_PARAMS = pltpu.CompilerParams(
    needs_layout_passes=False, use_tc_tiling_on_sc=False
)
_MESH = plsc.VectorSubcoreMesh(core_axis_name="c", subcore_axis_name="s")


def _wid():
    return lax.axis_index("s") * NC + lax.axis_index("c")


def _hist_body(shift, keys_hbm, counts_hbm, hist, kb):
    w = _wid()
    woff = w * CHUNK
    zeros = jnp.zeros((16,), jnp.int32)
    ones = jnp.ones((16,), jnp.int32)

    def z(i, _):
        hist[pl.ds(i * 16, 16)] = zeros
        return 0

    lax.fori_loop(0, NB // 16, z, 0)

    def sb_body(t, _):
        pltpu.sync_copy(keys_hbm.at[pl.ds(woff + t * SB, SB)], kb)

        def v_body(q, _):
            k = kb[pl.ds(q * 16, 16)]
            d = (k >> shift) & (NB - 1)
            plsc.addupdate_scatter(hist, [d], ones)
            return 0

        lax.fori_loop(0, SB // 16, v_body, 0)
        return 0

    lax.fori_loop(0, NSB, sb_body, 0)
    pltpu.sync_copy(hist, counts_hbm.at[pl.ds(w * NB, NB)])


def _make_hist(shift):
    return functools.partial(
        pl.kernel,
        out_type=jax.ShapeDtypeStruct((NW * NB,), jnp.int32),
        mesh=_MESH,
        scratch_types=[
            pltpu.VMEM((NB,), jnp.int32),
            pltpu.VMEM((SB,), jnp.int32),
        ],
        compiler_params=_SC_PARAMS,
    )(functools.partial(_hist_body, shift))


WSLICE = NP // 16


def _scat_body(shift, keys_hbm, counts_hbm, t_out,
               counts_v, acc, base_r, kb, vb, db, zb, table, sem):
    w = _wid()
    woff = w * CHUNK
    zeros = jnp.zeros((16,), jnp.int32)
    ones = jnp.ones((16,), jnp.int32)

    pltpu.sync_copy(counts_hbm, counts_v)

    def z(i, _):
        acc[pl.ds(i * 16, 16)] = zeros
        base_r[pl.ds(i * 16, 16)] = zeros
        return 0

    lax.fori_loop(0, NB // 16, z, 0)

    def tot_w(wp, _):
        def tot_i(i, _):
            s = pl.ds(i * 16, 16)
            acc[s] = acc[s] + counts_v[pl.ds(wp * NB + i * 16, 16)]
            return 0

        lax.fori_loop(0, NB // 16, tot_i, 0)
        return 0

    lax.fori_loop(0, NW, tot_w, 0)

    def pre_w(wp, _):
        def pre_i(i, _):
            s = pl.ds(i * 16, 16)
            base_r[s] = base_r[s] + counts_v[pl.ds(wp * NB + i * 16, 16)]
            return 0

        lax.fori_loop(0, NB // 16, pre_i, 0)
        return 0

    lax.fori_loop(0, w, pre_w, 0)

    def scan_i(i, c):
        s = pl.ds(i * 16, 16)
        v = acc[s]
        cs = plsc.cumsum(v)
        base_r[s] = base_r[s] + (cs - v) + c
        return c + jnp.sum(v)

    lax.fori_loop(0, NB // 16, scan_i, jnp.int32(0))

    sid = lax.axis_index("s")

    def zz(i, _):
        zb[pl.ds(i * 16, 16)] = zeros
        return 0

    lax.fori_loop(0, SB // 16, zz, 0)

    def zt(i, _):
        pltpu.sync_copy(zb, table.at[pl.ds(sid * WSLICE + i * SB, SB)])
        return 0

    lax.fori_loop(0, WSLICE // SB, zt, 0)
    plsc.subcore_barrier()

    def sb_body(t, _):
        pltpu.sync_copy(keys_hbm.at[pl.ds(woff + t * SB, SB)], kb)

        def v_body(q, _):
            k = kb[pl.ds(q * 16, 16)]
            d = (k >> shift) & (NB - 1)
            g = plsc.load_gather(base_r, [d])
            cnt, _unused = plsc.scan_count(d)
            plsc.addupdate_scatter(base_r, [d], ones)
            db[pl.ds(q * 16, 16)] = g + cnt - ones
            vb[pl.ds(q * 16, 16)] = (woff + t * SB + q * 16 + 1) + lax.iota(
                jnp.int32, 16)
            return 0

        lax.fori_loop(0, SB // 16, v_body, 0)
        pltpu.async_copy(vb, table.at[db], sem).wait()
        return 0

    lax.fori_loop(0, NSB, sb_body, 0)
    plsc.subcore_barrier()

    cid = lax.axis_index("c")
    pltpu.sync_copy(table.at[pl.ds(sid * WSLICE, WSLICE)],
                    t_out.at[pl.ds(cid * NP + sid * WSLICE, WSLICE)])


def _make_scat(shift):
    return functools.partial(
        pl.kernel,
        out_type=jax.ShapeDtypeStruct((2 * NP,), jnp.int32),
        mesh=_MESH,
        scratch_types=[
            pltpu.VMEM((NW * NB,), jnp.int32),
            pltpu.VMEM((NB,), jnp.int32),
            pltpu.VMEM((NB,), jnp.int32),
            pltpu.VMEM((SB,), jnp.int32),
            pltpu.VMEM((SB,), jnp.int32),
            pltpu.VMEM((SB,), jnp.int32),
            pltpu.VMEM((SB,), jnp.int32),
            pltpu.VMEM_SHARED((NP,), jnp.int32),
            pltpu.SemaphoreType.DMA,
        ],
        compiler_params=_SC_PARAMS,
    )(functools.partial(_scat_body, shift))


def _apply_body(shift, t_hbm, keys_hbm, keys_out, inv_out, counts_hbm,
                b0, b1, ivb, kb, hist, sem):
    w = _wid()
    woff = w * CHUNK
    zeros = jnp.zeros((16,), jnp.int32)
    ones = jnp.ones((16,), jnp.int32)

    def z(i, _):
        hist[pl.ds(i * 16, 16)] = zeros
        return 0

    lax.fori_loop(0, NB // 16, z, 0)

    def sb_body(t, _):
        pltpu.sync_copy(t_hbm.at[pl.ds(woff + t * SB, SB)], b0)
        pltpu.sync_copy(t_hbm.at[pl.ds(NP + woff + t * SB, SB)], b1)

        def inv_body(q, _):
            s = pl.ds(q * 16, 16)
            ivb[s] = b0[s] + b1[s] - ones
            return 0

        lax.fori_loop(0, SB // 16, inv_body, 0)
        pltpu.sync_copy(ivb, inv_out.at[pl.ds(woff + t * SB, SB)])
        pltpu.async_copy(keys_hbm.at[ivb], kb, sem).wait()

        def h_body(q, _):
            k = kb[pl.ds(q * 16, 16)]
            d = (k >> shift) & (NB - 1)
            plsc.addupdate_scatter(hist, [d], ones)
            return 0

        lax.fori_loop(0, SB // 16, h_body, 0)
        pltpu.sync_copy(kb, keys_out.at[pl.ds(woff + t * SB, SB)])
        return 0

    lax.fori_loop(0, NSB, sb_body, 0)
    pltpu.sync_copy(hist, counts_hbm.at[pl.ds(w * NB, NB)])


def _make_apply(next_shift):
    return functools.partial(
        pl.kernel,
        out_type=(
            jax.ShapeDtypeStruct((NP,), jnp.int32),
            jax.ShapeDtypeStruct((NP,), jnp.int32),
            jax.ShapeDtypeStruct((NW * NB,), jnp.int32),
        ),
        mesh=_MESH,
        scratch_types=[
            pltpu.VMEM((SB,), jnp.int32),
            pltpu.VMEM((SB,), jnp.int32),
            pltpu.VMEM((SB,), jnp.int32),
            pltpu.VMEM((SB,), jnp.int32),
            pltpu.VMEM((NB,), jnp.int32),
            pltpu.SemaphoreType.DMA,
        ],
        compiler_params=_SC_PARAMS,
    )(functools.partial(_apply_body, next_shift))


def _final_body(t_hbm, keys_hbm, inv1_hbm, inv2_hbm, pcf_hbm, colf_hbm,
                codes_out, pco_out, colo_out,
                b0, b1, ivb, i2b, sxb, kb, x3b, pcb, colb, sem):
    w = _wid()
    woff = w * CHUNK
    ones = jnp.ones((16,), jnp.int32)

    def sb_body(t, _):
        pltpu.sync_copy(t_hbm.at[pl.ds(woff + t * SB, SB)], b0)
        pltpu.sync_copy(t_hbm.at[pl.ds(NP + woff + t * SB, SB)], b1)

        def inv_body(q, _):
            s = pl.ds(q * 16, 16)
            ivb[s] = b0[s] + b1[s] - ones
            return 0

        lax.fori_loop(0, SB // 16, inv_body, 0)
        pltpu.async_copy(keys_hbm.at[ivb], kb, sem).wait()
        pltpu.sync_copy(kb, codes_out.at[pl.ds(woff + t * SB, SB)])
        pltpu.async_copy(inv2_hbm.at[ivb], i2b, sem).wait()
        pltpu.async_copy(inv1_hbm.at[i2b], sxb, sem).wait()

        def x3_body(q, _):
            sx = jnp.minimum(sxb[pl.ds(q * 16, 16)], N - 1) * 3
            pos = q * 48 + lax.iota(jnp.int32, 16) * 3
            plsc.store_scatter(x3b, [pos], sx)
            plsc.store_scatter(x3b, [pos + ones], sx + ones)
            plsc.store_scatter(x3b, [pos + ones + ones], sx + ones + ones)
            return 0

        lax.fori_loop(0, SB // 16, x3_body, 0)
        pltpu.async_copy(pcf_hbm.at[x3b], pcb, sem).wait()
        pltpu.async_copy(colf_hbm.at[x3b], colb, sem).wait()
        pltpu.sync_copy(pcb, pco_out.at[pl.ds(3 * (woff + t * SB), 3 * SB)])
        pltpu.sync_copy(colb, colo_out.at[pl.ds(3 * (woff + t * SB), 3 * SB)])
        return 0

    lax.fori_loop(0, NSB, sb_body, 0)


_final = functools.partial(
    pl.kernel,
    out_type=(
        jax.ShapeDtypeStruct((NP,), jnp.int32),
        jax.ShapeDtypeStruct((3 * NP,), jnp.float32),
        jax.ShapeDtypeStruct((3 * NP,), jnp.float32),
    ),
    mesh=_MESH,
    scratch_types=[
        pltpu.VMEM((SB,), jnp.int32),
        pltpu.VMEM((SB,), jnp.int32),
        pltpu.VMEM((SB,), jnp.int32),
        pltpu.VMEM((SB,), jnp.int32),
        pltpu.VMEM((SB,), jnp.int32),
        pltpu.VMEM((SB,), jnp.int32),
        pltpu.VMEM((3 * SB,), jnp.int32),
        pltpu.VMEM((3 * SB,), jnp.float32),
        pltpu.VMEM((3 * SB,), jnp.float32),
        pltpu.SemaphoreType.DMA,
    ],
    compiler_params=_SC_PARAMS,
)(_final_body)


def _gath_body(sidx_hbm, s0, s1, s2, s3, s4, s5,
               o0, o1, o2, o3, o4, o5,
               iv0, iv1,
               a0, a1, a2, a3, a4, a5,
               c0, c1, c2, c3, c4, c5,
               sem0, sem1):
    w = _wid()
    woff = w * CHUNK
    srcs = (s0, s1, s2, s3, s4, s5)
    outs = (o0, o1, o2, o3, o4, o5)
    bufs = ((a0, a1, a2, a3, a4, a5), (c0, c1, c2, c3, c4, c5))
    ivs = (iv0, iv1)
    sems = (sem0, sem1)

    def _issue(t, b):
        pltpu.sync_copy(sidx_hbm.at[pl.ds(woff + t * SB, SB)], ivs[b])
        for s, buf in zip(srcs, bufs[b]):
            pltpu.async_copy(s.at[ivs[b]], buf, sems[b])

    _issue(0, 0)
    for t in range(NSB):
        b = t % 2
        if t + 1 < NSB:
            _issue(t + 1, 1 - b)
        for s, buf in zip(srcs, bufs[b]):
            pltpu.make_async_copy(s.at[ivs[b]], buf, sems[b]).wait()
        for buf, o in zip(bufs[b], outs):
            pltpu.sync_copy(buf, o.at[pl.ds(woff + t * SB, SB)])


_gath = functools.partial(
    pl.kernel,
    out_type=tuple(
        jax.ShapeDtypeStruct((NP,), jnp.float32) for _ in range(6)
    ),
    mesh=_MESH,
    scratch_types=[pltpu.VMEM((SB,), jnp.int32) for _ in range(2)]
    + [pltpu.VMEM((SB,), jnp.float32) for _ in range(12)]
    + [pltpu.SemaphoreType.DMA, pltpu.SemaphoreType.DMA],
    compiler_params=_SC_PARAMS,
)(_gath_body)

_hist0 = _make_hist(0)
_scat0 = _make_scat(0)
_scat1 = _make_scat(10)
_scat2 = _make_scat(20)
_apply0 = _make_apply(10)
_apply1 = _make_apply(20)


def kernel(pointcloud, color):
    pct = pointcloud.T
    pcp = jnp.pad(pct, ((0, 0), (0, NP - N)))
    codes = _morton_codes_padded(pcp).reshape(NP)
    pcf = pointcloud.reshape(-1)
    colf = color.reshape(-1)

    c0 = _hist0(codes)
    t1 = _scat0(codes, c0)
    k1, inv1, c1 = _apply0(t1, codes)
    t2 = _scat1(k1, c1)
    k2, inv2, c2 = _apply1(t2, k1)
    t3 = _scat2(k2, c2)
    codes_s, pco_f, colo_f = _final(t3, k2, inv1, inv2, pcf, colf)
    pco = pco_f.reshape(NP, 3)
    colo = colo_f.reshape(NP, 3)
    return (pco[:N], colo[:N], codes_s[:N].astype(jnp.uint32))


# speedup vs baseline: 7.2499x; 7.2499x over previous
import functools

import jax
import jax.numpy as jnp
from jax import lax
from jax.experimental import pallas as pl
from jax.experimental.pallas import tpu as pltpu
from jax.experimental.pallas import tpu_sc as plsc

N = 1_000_000
NP = 1_003_520
NW = 32
NC = 2
CHUNK = NP // NW
NB = 1024
SB = 6272
NSB = CHUNK // SB
ROWS = SB // 128
PAD_CODE = (1 << 30) - 1


C = 6272
G = NP // C


def _expand3(v):
    x = v
    x = (x | (x << 16)) & 0x30000FF
    x = (x | (x << 8)) & 0x300F00F
    x = (x | (x << 4)) & 0x30C30C3
    x = (x | (x << 2)) & 0x9249249
    return x


def _morton_body(pc_ref, codes_ref, mm_ref):
    ph = pl.program_id(0)
    g = pl.program_id(1)

    @pl.when(ph == 0)
    def _():
        blk = pc_ref[...]
        bmin = jnp.min(blk, axis=1, keepdims=True)
        bmax = jnp.max(blk, axis=1, keepdims=True)
        prev_min = jnp.where(g == 0, jnp.full_like(bmin, jnp.inf), mm_ref[:, 0:1])
        prev_max = jnp.where(g == 0, jnp.full_like(bmax, -jnp.inf), mm_ref[:, 1:2])
        mm_ref[:, 0:1] = jnp.minimum(prev_min, bmin)
        mm_ref[:, 1:2] = jnp.maximum(prev_max, bmax)

    @pl.when(ph == 1)
    def _():
        blk = pc_ref[...]
        bmin = mm_ref[:, 0:1]
        bmax = mm_ref[:, 1:2]
        scale = jnp.float32(1023) / (bmax - bmin + jnp.float32(1e-7))
        q = jnp.floor((blk - bmin) * scale).astype(jnp.int32)
        q = jnp.minimum(q, 1023)
        e = _expand3(q)
        code = (e[0:1, :] << 2) | (e[1:2, :] << 1) | e[2:3, :]
        pos = g * C + lax.broadcasted_iota(jnp.int32, (1, C), 1)
        codes_ref[...] = jnp.where(pos < N, code, PAD_CODE)


def _morton_codes_padded(pcp):
    return pl.pallas_call(
        _morton_body,
        grid=(2, G),
        in_specs=[pl.BlockSpec((3, C), lambda ph, g: (0, g))],
        out_specs=pl.BlockSpec((1, C), lambda ph, g: (0, g)),
        out_shape=jax.ShapeDtypeStruct((1, NP), jnp.int32),
        scratch_shapes=[pltpu.VMEM((3, 2), jnp.float32)],
    )(pcp)


_SC_PARAMS = pltpu.CompilerParams(
    needs_layout_passes=False, use_tc_tiling_on_sc=False
)
_MESH = plsc.VectorSubcoreMesh(core_axis_name="c", subcore_axis_name="s")


def _wid():
    return lax.axis_index("s") * NC + lax.axis_index("c")


def _hist_body(shift, keys_hbm, counts_hbm, hist, kb):
    w = _wid()
    woff = w * CHUNK
    zeros = jnp.zeros((16,), jnp.int32)
    ones = jnp.ones((16,), jnp.int32)

    def z(i, _):
        hist[pl.ds(i * 16, 16)] = zeros
        return 0

    lax.fori_loop(0, NB // 16, z, 0)

    def sb_body(t, _):
        pltpu.sync_copy(keys_hbm.at[pl.ds(woff + t * SB, SB)], kb)

        def v_body(q, _):
            k = kb[pl.ds(q * 16, 16)]
            d = (k >> shift) & (NB - 1)
            plsc.addupdate_scatter(hist, [d], ones)
            return 0

        lax.fori_loop(0, SB // 16, v_body, 0)
        return 0

    lax.fori_loop(0, NSB, sb_body, 0)
    pltpu.sync_copy(hist, counts_hbm.at[pl.ds(w * NB, NB)])


def _make_hist(shift):
    return functools.partial(
        pl.kernel,
        out_type=jax.ShapeDtypeStruct((NW * NB,), jnp.int32),
        mesh=_MESH,
        scratch_types=[
            pltpu.VMEM((NB,), jnp.int32),
            pltpu.VMEM((SB,), jnp.int32),
        ],
        compiler_params=_SC_PARAMS,
    )(functools.partial(_hist_body, shift))


WSLICE = NP // 16


def _scat_body(shift, keys_hbm, counts_hbm, t_out,
               counts_v, acc, base_r, kb, vb, db, zb, table, sem):
    w = _wid()
    woff = w * CHUNK
    zeros = jnp.zeros((16,), jnp.int32)
    ones = jnp.ones((16,), jnp.int32)

    pltpu.sync_copy(counts_hbm, counts_v)

    def z(i, _):
        acc[pl.ds(i * 16, 16)] = zeros
        base_r[pl.ds(i * 16, 16)] = zeros
        return 0

    lax.fori_loop(0, NB // 16, z, 0)

    def tot_w(wp, _):
        def tot_i(i, _):
            s = pl.ds(i * 16, 16)
            acc[s] = acc[s] + counts_v[pl.ds(wp * NB + i * 16, 16)]
            return 0

        lax.fori_loop(0, NB // 16, tot_i, 0)
        return 0

    lax.fori_loop(0, NW, tot_w, 0)

    def pre_w(wp, _):
        def pre_i(i, _):
            s = pl.ds(i * 16, 16)
            base_r[s] = base_r[s] + counts_v[pl.ds(wp * NB + i * 16, 16)]
            return 0

        lax.fori_loop(0, NB // 16, pre_i, 0)
        return 0

    lax.fori_loop(0, w, pre_w, 0)

    def scan_i(i, c):
        s = pl.ds(i * 16, 16)
        v = acc[s]
        cs = plsc.cumsum(v)
        base_r[s] = base_r[s] + (cs - v) + c
        return c + jnp.sum(v)

    lax.fori_loop(0, NB // 16, scan_i, jnp.int32(0))

    sid = lax.axis_index("s")

    def zz(i, _):
        zb[pl.ds(i * 16, 16)] = zeros
        return 0

    lax.fori_loop(0, SB // 16, zz, 0)

    def zt(i, _):
        pltpu.sync_copy(zb, table.at[pl.ds(sid * WSLICE + i * SB, SB)])
        return 0

    lax.fori_loop(0, WSLICE // SB, zt, 0)
    plsc.subcore_barrier()

    def sb_body(t, _):
        pltpu.sync_copy(keys_hbm.at[pl.ds(woff + t * SB, SB)], kb)

        def v_body(q, _):
            k = kb[pl.ds(q * 16, 16)]
            d = (k >> shift) & (NB - 1)
            g = plsc.load_gather(base_r, [d])
            cnt, _unused = plsc.scan_count(d)
            plsc.addupdate_scatter(base_r, [d], ones)
            db[pl.ds(q * 16, 16)] = g + cnt - ones
            vb[pl.ds(q * 16, 16)] = (woff + t * SB + q * 16 + 1) + lax.iota(
                jnp.int32, 16)
            return 0

        lax.fori_loop(0, SB // 16, v_body, 0)
        pltpu.async_copy(vb, table.at[db], sem).wait()
        return 0

    lax.fori_loop(0, NSB, sb_body, 0)
    plsc.subcore_barrier()

    cid = lax.axis_index("c")
    pltpu.sync_copy(table.at[pl.ds(sid * WSLICE, WSLICE)],
                    t_out.at[pl.ds(cid * NP + sid * WSLICE, WSLICE)])


def _make_scat(shift):
    return functools.partial(
        pl.kernel,
        out_type=jax.ShapeDtypeStruct((2 * NP,), jnp.int32),
        mesh=_MESH,
        scratch_types=[
            pltpu.VMEM((NW * NB,), jnp.int32),
            pltpu.VMEM((NB,), jnp.int32),
            pltpu.VMEM((NB,), jnp.int32),
            pltpu.VMEM((SB,), jnp.int32),
            pltpu.VMEM((SB,), jnp.int32),
            pltpu.VMEM((SB,), jnp.int32),
            pltpu.VMEM((SB,), jnp.int32),
            pltpu.VMEM_SHARED((NP,), jnp.int32),
            pltpu.SemaphoreType.DMA,
        ],
        compiler_params=_SC_PARAMS,
    )(functools.partial(_scat_body, shift))


def _apply_body(shift, t_hbm, keys_hbm, keys_out, inv_out, counts_hbm,
                b0, b1, ivb, kb, hist, sem):
    w = _wid()
    woff = w * CHUNK
    zeros = jnp.zeros((16,), jnp.int32)
    ones = jnp.ones((16,), jnp.int32)

    def z(i, _):
        hist[pl.ds(i * 16, 16)] = zeros
        return 0

    lax.fori_loop(0, NB // 16, z, 0)

    def sb_body(t, _):
        pltpu.sync_copy(t_hbm.at[pl.ds(woff + t * SB, SB)], b0)
        pltpu.sync_copy(t_hbm.at[pl.ds(NP + woff + t * SB, SB)], b1)

        def inv_body(q, _):
            s = pl.ds(q * 16, 16)
            ivb[s] = b0[s] + b1[s] - ones
            return 0

        lax.fori_loop(0, SB // 16, inv_body, 0)
        pltpu.sync_copy(ivb, inv_out.at[pl.ds(woff + t * SB, SB)])
        pltpu.async_copy(keys_hbm.at[ivb], kb, sem).wait()

        def h_body(q, _):
            k = kb[pl.ds(q * 16, 16)]
            d = (k >> shift) & (NB - 1)
            plsc.addupdate_scatter(hist, [d], ones)
            return 0

        lax.fori_loop(0, SB // 16, h_body, 0)
        pltpu.sync_copy(kb, keys_out.at[pl.ds(woff + t * SB, SB)])
        return 0

    lax.fori_loop(0, NSB, sb_body, 0)
    pltpu.sync_copy(hist, counts_hbm.at[pl.ds(w * NB, NB)])


def _make_apply(next_shift):
    return functools.partial(
        pl.kernel,
        out_type=(
            jax.ShapeDtypeStruct((NP,), jnp.int32),
            jax.ShapeDtypeStruct((NP,), jnp.int32),
            jax.ShapeDtypeStruct((NW * NB,), jnp.int32),
        ),
        mesh=_MESH,
        scratch_types=[
            pltpu.VMEM((SB,), jnp.int32),
            pltpu.VMEM((SB,), jnp.int32),
            pltpu.VMEM((SB,), jnp.int32),
            pltpu.VMEM((SB,), jnp.int32),
            pltpu.VMEM((NB,), jnp.int32),
            pltpu.SemaphoreType.DMA,
        ],
        compiler_params=_SC_PARAMS,
    )(functools.partial(_apply_body, next_shift))


def _final_body(t_hbm, keys_hbm, inv1_hbm, inv2_hbm,
                s0, s1, s2, s3, s4, s5,
                codes_out, o0, o1, o2, o3, o4, o5,
                b0, b1, ivb, i2b, sxb, kb,
                g0, g1, g2, g3, g4, g5, sem):
    w = _wid()
    woff = w * CHUNK
    ones = jnp.ones((16,), jnp.int32)
    srcs = (s0, s1, s2, s3, s4, s5)
    outs = (o0, o1, o2, o3, o4, o5)
    bufs = (g0, g1, g2, g3, g4, g5)

    def sb_body(t, _):
        pltpu.sync_copy(t_hbm.at[pl.ds(woff + t * SB, SB)], b0)
        pltpu.sync_copy(t_hbm.at[pl.ds(NP + woff + t * SB, SB)], b1)

        def inv_body(q, _):
            s = pl.ds(q * 16, 16)
            ivb[s] = b0[s] + b1[s] - ones
            return 0

        lax.fori_loop(0, SB // 16, inv_body, 0)
        pltpu.async_copy(keys_hbm.at[ivb], kb, sem).wait()
        pltpu.sync_copy(kb, codes_out.at[pl.ds(woff + t * SB, SB)])
        pltpu.async_copy(inv2_hbm.at[ivb], i2b, sem).wait()
        pltpu.async_copy(inv1_hbm.at[i2b], sxb, sem).wait()

        def clamp_body(q, _):
            s = pl.ds(q * 16, 16)
            sxb[s] = jnp.minimum(sxb[s], N - 1)
            return 0

        lax.fori_loop(0, SB // 16, clamp_body, 0)
        descs = [pltpu.async_copy(s.at[sxb], b, sem)
                 for s, b in zip(srcs, bufs)]
        for d in descs:
            d.wait()
        for b, o in zip(bufs, outs):
            pltpu.sync_copy(b, o.at[pl.ds(woff + t * SB, SB)])
        return 0

    lax.fori_loop(0, NSB, sb_body, 0)


_final = functools.partial(
    pl.kernel,
    out_type=tuple(
        [jax.ShapeDtypeStruct((NP,), jnp.int32)]
        + [jax.ShapeDtypeStruct((NP,), jnp.float32) for _ in range(6)]
    ),
    mesh=_MESH,
    scratch_types=[pltpu.VMEM((SB,), jnp.int32) for _ in range(6)]
    + [pltpu.VMEM((SB,), jnp.float32) for _ in range(6)]
    + [pltpu.SemaphoreType.DMA],
    compiler_params=_SC_PARAMS,
)(_final_body)


def _gath_body(sidx_hbm, s0, s1, s2, s3, s4, s5,
               o0, o1, o2, o3, o4, o5,
               iv0, iv1,
               a0, a1, a2, a3, a4, a5,
               c0, c1, c2, c3, c4, c5,
               sem0, sem1):
    w = _wid()
    woff = w * CHUNK
    srcs = (s0, s1, s2, s3, s4, s5)
    outs = (o0, o1, o2, o3, o4, o5)
    bufs = ((a0, a1, a2, a3, a4, a5), (c0, c1, c2, c3, c4, c5))
    ivs = (iv0, iv1)
    sems = (sem0, sem1)

    def _issue(t, b):
        pltpu.sync_copy(sidx_hbm.at[pl.ds(woff + t * SB, SB)], ivs[b])
        for s, buf in zip(srcs, bufs[b]):
            pltpu.async_copy(s.at[ivs[b]], buf, sems[b])

    _issue(0, 0)
    for t in range(NSB):
        b = t % 2
        if t + 1 < NSB:
            _issue(t + 1, 1 - b)
        for s, buf in zip(srcs, bufs[b]):
            pltpu.make_async_copy(s.at[ivs[b]], buf, sems[b]).wait()
        for buf, o in zip(bufs[b], outs):
            pltpu.sync_copy(buf, o.at[pl.ds(woff + t * SB, SB)])


_gath = functools.partial(
    pl.kernel,
    out_type=tuple(
        jax.ShapeDtypeStruct((NP,), jnp.float32) for _ in range(6)
    ),
    mesh=_MESH,
    scratch_types=[pltpu.VMEM((SB,), jnp.int32) for _ in range(2)]
    + [pltpu.VMEM((SB,), jnp.float32) for _ in range(12)]
    + [pltpu.SemaphoreType.DMA, pltpu.SemaphoreType.DMA],
    compiler_params=_SC_PARAMS,
)(_gath_body)

_hist0 = _make_hist(0)
_scat0 = _make_scat(0)
_scat1 = _make_scat(10)
_scat2 = _make_scat(20)
_apply0 = _make_apply(10)
_apply1 = _make_apply(20)


def kernel(pointcloud, color):
    pct = pointcloud.T
    pcp = jnp.pad(pct, ((0, 0), (0, NP - N)))
    codes = _morton_codes_padded(pcp).reshape(NP)
    colt = color.T

    c0 = _hist0(codes)
    t1 = _scat0(codes, c0)
    k1, inv1, c1 = _apply0(t1, codes)
    t2 = _scat1(k1, c1)
    k2, inv2, c2 = _apply1(t2, k1)
    t3 = _scat2(k2, c2)
    codes_s, ox, oy, oz, orr, og, ob = _final(
        t3, k2, inv1, inv2, pct[0], pct[1], pct[2], colt[0], colt[1], colt[2])
    pco = jnp.stack([ox, oy, oz], axis=1)
    colo = jnp.stack([orr, og, ob], axis=1)
    return (pco[:N], colo[:N], codes_s[:N].astype(jnp.uint32))


# speedup vs baseline: 7.3000x; 1.0069x over previous
import functools

import jax
import jax.numpy as jnp
from jax import lax
from jax.experimental import pallas as pl
from jax.experimental.pallas import tpu as pltpu
from jax.experimental.pallas import tpu_sc as plsc

N = 1_000_000
NP = 1_003_520
NW = 32
NC = 2
CHUNK = NP // NW
NB = 1024
SB = 6272
NSB = CHUNK // SB
ROWS = SB // 128
PAD_CODE = (1 << 30) - 1


C = 6272
G = NP // C


def _expand3(v):
    x = v
    x = (x | (x << 16)) & 0x30000FF
    x = (x | (x << 8)) & 0x300F00F
    x = (x | (x << 4)) & 0x30C30C3
    x = (x | (x << 2)) & 0x9249249
    return x


def _morton_body(pc_ref, codes_ref, mm_ref):
    ph = pl.program_id(0)
    g = pl.program_id(1)

    @pl.when(ph == 0)
    def _():
        blk = pc_ref[...]
        bmin = jnp.min(blk, axis=1, keepdims=True)
        bmax = jnp.max(blk, axis=1, keepdims=True)
        prev_min = jnp.where(g == 0, jnp.full_like(bmin, jnp.inf), mm_ref[:, 0:1])
        prev_max = jnp.where(g == 0, jnp.full_like(bmax, -jnp.inf), mm_ref[:, 1:2])
        mm_ref[:, 0:1] = jnp.minimum(prev_min, bmin)
        mm_ref[:, 1:2] = jnp.maximum(prev_max, bmax)

    @pl.when(ph == 1)
    def _():
        blk = pc_ref[...]
        bmin = mm_ref[:, 0:1]
        bmax = mm_ref[:, 1:2]
        scale = jnp.float32(1023) / (bmax - bmin + jnp.float32(1e-7))
        q = jnp.floor((blk - bmin) * scale).astype(jnp.int32)
        q = jnp.minimum(q, 1023)
        e = _expand3(q)
        code = (e[0:1, :] << 2) | (e[1:2, :] << 1) | e[2:3, :]
        pos = g * C + lax.broadcasted_iota(jnp.int32, (1, C), 1)
        codes_ref[...] = jnp.where(pos < N, code, PAD_CODE)


def _morton_codes_padded(pcp):
    return pl.pallas_call(
        _morton_body,
        grid=(2, G),
        in_specs=[pl.BlockSpec((3, C), lambda ph, g: (0, g))],
        out_specs=pl.BlockSpec((1, C), lambda ph, g: (0, g)),
        out_shape=jax.ShapeDtypeStruct((1, NP), jnp.int32),
        scratch_shapes=[pltpu.VMEM((3, 2), jnp.float32)],
    )(pcp)


_SC_PARAMS = pltpu.CompilerParams(
    needs_layout_passes=False, use_tc_tiling_on_sc=False
)
_MESH = plsc.VectorSubcoreMesh(core_axis_name="c", subcore_axis_name="s")


def _wid():
    return lax.axis_index("s") * NC + lax.axis_index("c")


def _hist_body(shift, keys_hbm, counts_hbm, hist, kb):
    w = _wid()
    woff = w * CHUNK
    zeros = jnp.zeros((16,), jnp.int32)
    ones = jnp.ones((16,), jnp.int32)

    def z(i, _):
        hist[pl.ds(i * 16, 16)] = zeros
        return 0

    lax.fori_loop(0, NB // 16, z, 0)

    def sb_body(t, _):
        pltpu.sync_copy(keys_hbm.at[pl.ds(woff + t * SB, SB)], kb)

        def v_body(q, _):
            k = kb[pl.ds(q * 16, 16)]
            d = (k >> shift) & (NB - 1)
            plsc.addupdate_scatter(hist, [d], ones)
            return 0

        lax.fori_loop(0, SB // 16, v_body, 0)
        return 0

    lax.fori_loop(0, NSB, sb_body, 0)
    pltpu.sync_copy(hist, counts_hbm.at[pl.ds(w * NB, NB)])


def _make_hist(shift):
    return functools.partial(
        pl.kernel,
        out_type=jax.ShapeDtypeStruct((NW * NB,), jnp.int32),
        mesh=_MESH,
        scratch_types=[
            pltpu.VMEM((NB,), jnp.int32),
            pltpu.VMEM((SB,), jnp.int32),
        ],
        compiler_params=_SC_PARAMS,
    )(functools.partial(_hist_body, shift))


WSLICE = NP // 16


def _scat_body(shift, keys_hbm, counts_hbm, t_out,
               counts_v, acc, base_r, kb, vb, db, zb, table, sem):
    w = _wid()
    woff = w * CHUNK
    zeros = jnp.zeros((16,), jnp.int32)
    ones = jnp.ones((16,), jnp.int32)

    pltpu.sync_copy(counts_hbm, counts_v)

    def z(i, _):
        acc[pl.ds(i * 16, 16)] = zeros
        base_r[pl.ds(i * 16, 16)] = zeros
        return 0

    lax.fori_loop(0, NB // 16, z, 0)

    def tot_w(wp, _):
        def tot_i(i, _):
            s = pl.ds(i * 16, 16)
            acc[s] = acc[s] + counts_v[pl.ds(wp * NB + i * 16, 16)]
            return 0

        lax.fori_loop(0, NB // 16, tot_i, 0)
        return 0

    lax.fori_loop(0, NW, tot_w, 0)

    def pre_w(wp, _):
        def pre_i(i, _):
            s = pl.ds(i * 16, 16)
            base_r[s] = base_r[s] + counts_v[pl.ds(wp * NB + i * 16, 16)]
            return 0

        lax.fori_loop(0, NB // 16, pre_i, 0)
        return 0

    lax.fori_loop(0, w, pre_w, 0)

    def scan_i(i, c):
        s = pl.ds(i * 16, 16)
        v = acc[s]
        cs = plsc.cumsum(v)
        base_r[s] = base_r[s] + (cs - v) + c
        return c + jnp.sum(v)

    lax.fori_loop(0, NB // 16, scan_i, jnp.int32(0))

    sid = lax.axis_index("s")

    def zz(i, _):
        zb[pl.ds(i * 16, 16)] = zeros
        return 0

    lax.fori_loop(0, SB // 16, zz, 0)

    def zt(i, _):
        pltpu.sync_copy(zb, table.at[pl.ds(sid * WSLICE + i * SB, SB)])
        return 0

    lax.fori_loop(0, WSLICE // SB, zt, 0)
    plsc.subcore_barrier()

    def sb_body(t, _):
        pltpu.sync_copy(keys_hbm.at[pl.ds(woff + t * SB, SB)], kb)

        def v_body(q, _):
            k = kb[pl.ds(q * 16, 16)]
            d = (k >> shift) & (NB - 1)
            g = plsc.load_gather(base_r, [d])
            cnt, _unused = plsc.scan_count(d)
            plsc.addupdate_scatter(base_r, [d], ones)
            db[pl.ds(q * 16, 16)] = g + cnt - ones
            vb[pl.ds(q * 16, 16)] = (woff + t * SB + q * 16 + 1) + lax.iota(
                jnp.int32, 16)
            return 0

        lax.fori_loop(0, SB // 16, v_body, 0)
        pltpu.async_copy(vb, table.at[db], sem).wait()
        return 0

    lax.fori_loop(0, NSB, sb_body, 0)
    plsc.subcore_barrier()

    cid = lax.axis_index("c")
    pltpu.sync_copy(table.at[pl.ds(sid * WSLICE, WSLICE)],
                    t_out.at[pl.ds(cid * NP + sid * WSLICE, WSLICE)])


def _make_scat(shift):
    return functools.partial(
        pl.kernel,
        out_type=jax.ShapeDtypeStruct((2 * NP,), jnp.int32),
        mesh=_MESH,
        scratch_types=[
            pltpu.VMEM((NW * NB,), jnp.int32),
            pltpu.VMEM((NB,), jnp.int32),
            pltpu.VMEM((NB,), jnp.int32),
            pltpu.VMEM((SB,), jnp.int32),
            pltpu.VMEM((SB,), jnp.int32),
            pltpu.VMEM((SB,), jnp.int32),
            pltpu.VMEM((SB,), jnp.int32),
            pltpu.VMEM_SHARED((NP,), jnp.int32),
            pltpu.SemaphoreType.DMA,
        ],
        compiler_params=_SC_PARAMS,
    )(functools.partial(_scat_body, shift))


def _apply_body(shift, t_hbm, keys_hbm, keys_out, inv_out, counts_hbm,
                b0, b1, ivb, kb, hist, sem):
    w = _wid()
    woff = w * CHUNK
    zeros = jnp.zeros((16,), jnp.int32)
    ones = jnp.ones((16,), jnp.int32)

    def z(i, _):
        hist[pl.ds(i * 16, 16)] = zeros
        return 0

    lax.fori_loop(0, NB // 16, z, 0)

    def sb_body(t, _):
        pltpu.sync_copy(t_hbm.at[pl.ds(woff + t * SB, SB)], b0)
        pltpu.sync_copy(t_hbm.at[pl.ds(NP + woff + t * SB, SB)], b1)

        def inv_body(q, _):
            s = pl.ds(q * 16, 16)
            ivb[s] = b0[s] + b1[s] - ones
            return 0

        lax.fori_loop(0, SB // 16, inv_body, 0)
        pltpu.sync_copy(ivb, inv_out.at[pl.ds(woff + t * SB, SB)])
        pltpu.async_copy(keys_hbm.at[ivb], kb, sem).wait()

        def h_body(q, _):
            k = kb[pl.ds(q * 16, 16)]
            d = (k >> shift) & (NB - 1)
            plsc.addupdate_scatter(hist, [d], ones)
            return 0

        lax.fori_loop(0, SB // 16, h_body, 0)
        pltpu.sync_copy(kb, keys_out.at[pl.ds(woff + t * SB, SB)])
        return 0

    lax.fori_loop(0, NSB, sb_body, 0)
    pltpu.sync_copy(hist, counts_hbm.at[pl.ds(w * NB, NB)])


def _make_apply(next_shift):
    return functools.partial(
        pl.kernel,
        out_type=(
            jax.ShapeDtypeStruct((NP,), jnp.int32),
            jax.ShapeDtypeStruct((NP,), jnp.int32),
            jax.ShapeDtypeStruct((NW * NB,), jnp.int32),
        ),
        mesh=_MESH,
        scratch_types=[
            pltpu.VMEM((SB,), jnp.int32),
            pltpu.VMEM((SB,), jnp.int32),
            pltpu.VMEM((SB,), jnp.int32),
            pltpu.VMEM((SB,), jnp.int32),
            pltpu.VMEM((NB,), jnp.int32),
            pltpu.SemaphoreType.DMA,
        ],
        compiler_params=_SC_PARAMS,
    )(functools.partial(_apply_body, next_shift))


def _final_body(t_hbm, keys_hbm, inv1_hbm, inv2_hbm,
                s0, s1, s2, s3, s4, s5,
                codes_out, o0, o1, o2, o3, o4, o5,
                b0, b1, ivb, i2b, sxb, kb,
                g0, g1, g2, g3, g4, g5, sem, sem2):
    w = _wid()
    woff = w * CHUNK
    ones = jnp.ones((16,), jnp.int32)
    srcs = (s0, s1, s2, s3, s4, s5)
    outs = (o0, o1, o2, o3, o4, o5)
    bufs = (g0, g1, g2, g3, g4, g5)

    def sb_body(t, _):
        pltpu.sync_copy(t_hbm.at[pl.ds(woff + t * SB, SB)], b0)
        pltpu.sync_copy(t_hbm.at[pl.ds(NP + woff + t * SB, SB)], b1)

        def inv_body(q, _):
            s = pl.ds(q * 16, 16)
            ivb[s] = b0[s] + b1[s] - ones
            return 0

        lax.fori_loop(0, SB // 16, inv_body, 0)
        dk = pltpu.async_copy(keys_hbm.at[ivb], kb, sem)
        d2 = pltpu.async_copy(inv2_hbm.at[ivb], i2b, sem2)
        d2.wait()
        ds_ = pltpu.async_copy(inv1_hbm.at[i2b], sxb, sem2)
        dk.wait()
        pltpu.sync_copy(kb, codes_out.at[pl.ds(woff + t * SB, SB)])
        ds_.wait()

        def clamp_body(q, _):
            s = pl.ds(q * 16, 16)
            sxb[s] = jnp.minimum(sxb[s], N - 1)
            return 0

        lax.fori_loop(0, SB // 16, clamp_body, 0)
        descs = [pltpu.async_copy(s.at[sxb], b, sem)
                 for s, b in zip(srcs, bufs)]
        for d in descs:
            d.wait()
        for b, o in zip(bufs, outs):
            pltpu.sync_copy(b, o.at[pl.ds(woff + t * SB, SB)])
        return 0

    lax.fori_loop(0, NSB, sb_body, 0)


_final = functools.partial(
    pl.kernel,
    out_type=tuple(
        [jax.ShapeDtypeStruct((NP,), jnp.int32)]
        + [jax.ShapeDtypeStruct((NP,), jnp.float32) for _ in range(6)]
    ),
    mesh=_MESH,
    scratch_types=[pltpu.VMEM((SB,), jnp.int32) for _ in range(6)]
    + [pltpu.VMEM((SB,), jnp.float32) for _ in range(6)]
    + [pltpu.SemaphoreType.DMA, pltpu.SemaphoreType.DMA],
    compiler_params=_SC_PARAMS,
)(_final_body)


def _gath_body(sidx_hbm, s0, s1, s2, s3, s4, s5,
               o0, o1, o2, o3, o4, o5,
               iv0, iv1,
               a0, a1, a2, a3, a4, a5,
               c0, c1, c2, c3, c4, c5,
               sem0, sem1):
    w = _wid()
    woff = w * CHUNK
    srcs = (s0, s1, s2, s3, s4, s5)
    outs = (o0, o1, o2, o3, o4, o5)
    bufs = ((a0, a1, a2, a3, a4, a5), (c0, c1, c2, c3, c4, c5))
    ivs = (iv0, iv1)
    sems = (sem0, sem1)

    def _issue(t, b):
        pltpu.sync_copy(sidx_hbm.at[pl.ds(woff + t * SB, SB)], ivs[b])
        for s, buf in zip(srcs, bufs[b]):
            pltpu.async_copy(s.at[ivs[b]], buf, sems[b])

    _issue(0, 0)
    for t in range(NSB):
        b = t % 2
        if t + 1 < NSB:
            _issue(t + 1, 1 - b)
        for s, buf in zip(srcs, bufs[b]):
            pltpu.make_async_copy(s.at[ivs[b]], buf, sems[b]).wait()
        for buf, o in zip(bufs[b], outs):
            pltpu.sync_copy(buf, o.at[pl.ds(woff + t * SB, SB)])


_gath = functools.partial(
    pl.kernel,
    out_type=tuple(
        jax.ShapeDtypeStruct((NP,), jnp.float32) for _ in range(6)
    ),
    mesh=_MESH,
    scratch_types=[pltpu.VMEM((SB,), jnp.int32) for _ in range(2)]
    + [pltpu.VMEM((SB,), jnp.float32) for _ in range(12)]
    + [pltpu.SemaphoreType.DMA, pltpu.SemaphoreType.DMA],
    compiler_params=_SC_PARAMS,
)(_gath_body)

_hist0 = _make_hist(0)
_scat0 = _make_scat(0)
_scat1 = _make_scat(10)
_scat2 = _make_scat(20)
_apply0 = _make_apply(10)
_apply1 = _make_apply(20)


def kernel(pointcloud, color):
    pct = pointcloud.T
    pcp = jnp.pad(pct, ((0, 0), (0, NP - N)))
    codes = _morton_codes_padded(pcp).reshape(NP)
    colt = color.T

    c0 = _hist0(codes)
    t1 = _scat0(codes, c0)
    k1, inv1, c1 = _apply0(t1, codes)
    t2 = _scat1(k1, c1)
    k2, inv2, c2 = _apply1(t2, k1)
    t3 = _scat2(k2, c2)
    codes_s, ox, oy, oz, orr, og, ob = _final(
        t3, k2, inv1, inv2, pct[0], pct[1], pct[2], colt[0], colt[1], colt[2])
    pco = jnp.stack([ox, oy, oz], axis=1)
    colo = jnp.stack([orr, og, ob], axis=1)
    return (pco[:N], colo[:N], codes_s[:N].astype(jnp.uint32))


# speedup vs baseline: 7.4374x; 1.0188x over previous
import functools

import jax
import jax.numpy as jnp
from jax import lax
from jax.experimental import pallas as pl
from jax.experimental.pallas import tpu as pltpu
from jax.experimental.pallas import tpu_sc as plsc

N = 1_000_000
NP = 1_003_520
NW = 32
NC = 2
CHUNK = NP // NW
NB = 1024
SB = 6272
NSB = CHUNK // SB
ROWS = SB // 128
PAD_CODE = (1 << 30) - 1


C = 6272
G = NP // C


def _expand3(v):
    x = v
    x = (x | (x << 16)) & 0x30000FF
    x = (x | (x << 8)) & 0x300F00F
    x = (x | (x << 4)) & 0x30C30C3
    x = (x | (x << 2)) & 0x9249249
    return x


def _morton_body(pc_ref, codes_ref, mm_ref):
    ph = pl.program_id(0)
    g = pl.program_id(1)

    @pl.when(ph == 0)
    def _():
        blk = pc_ref[...]
        bmin = jnp.min(blk, axis=1, keepdims=True)
        bmax = jnp.max(blk, axis=1, keepdims=True)
        prev_min = jnp.where(g == 0, jnp.full_like(bmin, jnp.inf), mm_ref[:, 0:1])
        prev_max = jnp.where(g == 0, jnp.full_like(bmax, -jnp.inf), mm_ref[:, 1:2])
        mm_ref[:, 0:1] = jnp.minimum(prev_min, bmin)
        mm_ref[:, 1:2] = jnp.maximum(prev_max, bmax)

    @pl.when(ph == 1)
    def _():
        blk = pc_ref[...]
        bmin = mm_ref[:, 0:1]
        bmax = mm_ref[:, 1:2]
        scale = jnp.float32(1023) / (bmax - bmin + jnp.float32(1e-7))
        q = jnp.floor((blk - bmin) * scale).astype(jnp.int32)
        q = jnp.minimum(q, 1023)
        e = _expand3(q)
        code = (e[0:1, :] << 2) | (e[1:2, :] << 1) | e[2:3, :]
        pos = g * C + lax.broadcasted_iota(jnp.int32, (1, C), 1)
        codes_ref[...] = jnp.where(pos < N, code, PAD_CODE)


def _morton_codes_padded(pcp):
    return pl.pallas_call(
        _morton_body,
        grid=(2, G),
        in_specs=[pl.BlockSpec((3, C), lambda ph, g: (0, g))],
        out_specs=pl.BlockSpec((1, C), lambda ph, g: (0, g)),
        out_shape=jax.ShapeDtypeStruct((1, NP), jnp.int32),
        scratch_shapes=[pltpu.VMEM((3, 2), jnp.float32)],
    )(pcp)


_SC_PARAMS = pltpu.CompilerParams(
    needs_layout_passes=False, use_tc_tiling_on_sc=False
)
_MESH = plsc.VectorSubcoreMesh(core_axis_name="c", subcore_axis_name="s")


def _wid():
    return lax.axis_index("s") * NC + lax.axis_index("c")


def _hist_body(shift, keys_hbm, counts_hbm, hist, kb):
    w = _wid()
    woff = w * CHUNK
    zeros = jnp.zeros((16,), jnp.int32)
    ones = jnp.ones((16,), jnp.int32)

    def z(i, _):
        hist[pl.ds(i * 16, 16)] = zeros
        return 0

    lax.fori_loop(0, NB // 16, z, 0)

    def sb_body(t, _):
        pltpu.sync_copy(keys_hbm.at[pl.ds(woff + t * SB, SB)], kb)

        def v_body(q, _):
            k = kb[pl.ds(q * 16, 16)]
            d = (k >> shift) & (NB - 1)
            plsc.addupdate_scatter(hist, [d], ones)
            return 0

        lax.fori_loop(0, SB // 16, v_body, 0)
        return 0

    lax.fori_loop(0, NSB, sb_body, 0)
    pltpu.sync_copy(hist, counts_hbm.at[pl.ds(w * NB, NB)])


def _make_hist(shift):
    return functools.partial(
        pl.kernel,
        out_type=jax.ShapeDtypeStruct((NW * NB,), jnp.int32),
        mesh=_MESH,
        scratch_types=[
            pltpu.VMEM((NB,), jnp.int32),
            pltpu.VMEM((SB,), jnp.int32),
        ],
        compiler_params=_SC_PARAMS,
    )(functools.partial(_hist_body, shift))


WSLICE = NP // 16


def _scat_body(shift, keys_hbm, counts_hbm, t_out,
               counts_v, acc, base_r, kb, vb, db, zb, table, sem):
    w = _wid()
    woff = w * CHUNK
    zeros = jnp.zeros((16,), jnp.int32)
    ones = jnp.ones((16,), jnp.int32)

    pltpu.sync_copy(counts_hbm, counts_v)

    def z(i, _):
        acc[pl.ds(i * 16, 16)] = zeros
        base_r[pl.ds(i * 16, 16)] = zeros
        return 0

    lax.fori_loop(0, NB // 16, z, 0)

    def tot_w(wp, _):
        @pl.when(wp == w)
        def _():
            def cp(i, _):
                s = pl.ds(i * 16, 16)
                base_r[s] = acc[s]
                return 0

            lax.fori_loop(0, NB // 16, cp, 0)

        def tot_i(i, _):
            for u in range(4):
                s = pl.ds(i * 64 + u * 16, 16)
                acc[s] = acc[s] + counts_v[pl.ds(wp * NB + i * 64 + u * 16, 16)]
            return 0

        lax.fori_loop(0, NB // 64, tot_i, 0)
        return 0

    lax.fori_loop(0, NW, tot_w, 0)

    def scan_i(i, c):
        s = pl.ds(i * 16, 16)
        v = acc[s]
        cs = plsc.cumsum(v)
        base_r[s] = base_r[s] + (cs - v) + c
        return c + jnp.sum(v)

    lax.fori_loop(0, NB // 16, scan_i, jnp.int32(0))

    sid = lax.axis_index("s")

    def zz(i, _):
        zb[pl.ds(i * 16, 16)] = zeros
        return 0

    lax.fori_loop(0, SB // 16, zz, 0)

    def zt(i, _):
        pltpu.sync_copy(zb, table.at[pl.ds(sid * WSLICE + i * SB, SB)])
        return 0

    lax.fori_loop(0, WSLICE // SB, zt, 0)
    plsc.subcore_barrier()

    def sb_body(t, _):
        pltpu.sync_copy(keys_hbm.at[pl.ds(woff + t * SB, SB)], kb)

        def v_body(q, _):
            k = kb[pl.ds(q * 16, 16)]
            d = (k >> shift) & (NB - 1)
            g = plsc.load_gather(base_r, [d])
            cnt, _unused = plsc.scan_count(d)
            plsc.addupdate_scatter(base_r, [d], ones)
            db[pl.ds(q * 16, 16)] = g + cnt - ones
            vb[pl.ds(q * 16, 16)] = (woff + t * SB + q * 16 + 1) + lax.iota(
                jnp.int32, 16)
            return 0

        lax.fori_loop(0, SB // 16, v_body, 0)
        pltpu.async_copy(vb, table.at[db], sem).wait()
        return 0

    lax.fori_loop(0, NSB, sb_body, 0)
    plsc.subcore_barrier()

    cid = lax.axis_index("c")
    pltpu.sync_copy(table.at[pl.ds(sid * WSLICE, WSLICE)],
                    t_out.at[pl.ds(cid * NP + sid * WSLICE, WSLICE)])


def _make_scat(shift):
    return functools.partial(
        pl.kernel,
        out_type=jax.ShapeDtypeStruct((2 * NP,), jnp.int32),
        mesh=_MESH,
        scratch_types=[
            pltpu.VMEM((NW * NB,), jnp.int32),
            pltpu.VMEM((NB,), jnp.int32),
            pltpu.VMEM((NB,), jnp.int32),
            pltpu.VMEM((SB,), jnp.int32),
            pltpu.VMEM((SB,), jnp.int32),
            pltpu.VMEM((SB,), jnp.int32),
            pltpu.VMEM((SB,), jnp.int32),
            pltpu.VMEM_SHARED((NP,), jnp.int32),
            pltpu.SemaphoreType.DMA,
        ],
        compiler_params=_SC_PARAMS,
    )(functools.partial(_scat_body, shift))


def _apply_body(shift, t_hbm, keys_hbm, keys_out, inv_out, counts_hbm,
                b0, b1, ivb, kb, hist, sem):
    w = _wid()
    woff = w * CHUNK
    zeros = jnp.zeros((16,), jnp.int32)
    ones = jnp.ones((16,), jnp.int32)

    def z(i, _):
        hist[pl.ds(i * 16, 16)] = zeros
        return 0

    lax.fori_loop(0, NB // 16, z, 0)

    def sb_body(t, _):
        pltpu.sync_copy(t_hbm.at[pl.ds(woff + t * SB, SB)], b0)
        pltpu.sync_copy(t_hbm.at[pl.ds(NP + woff + t * SB, SB)], b1)

        def inv_body(q, _):
            s = pl.ds(q * 16, 16)
            ivb[s] = b0[s] + b1[s] - ones
            return 0

        lax.fori_loop(0, SB // 16, inv_body, 0)
        pltpu.sync_copy(ivb, inv_out.at[pl.ds(woff + t * SB, SB)])
        pltpu.async_copy(keys_hbm.at[ivb], kb, sem).wait()

        def h_body(q, _):
            k = kb[pl.ds(q * 16, 16)]
            d = (k >> shift) & (NB - 1)
            plsc.addupdate_scatter(hist, [d], ones)
            return 0

        lax.fori_loop(0, SB // 16, h_body, 0)
        pltpu.sync_copy(kb, keys_out.at[pl.ds(woff + t * SB, SB)])
        return 0

    lax.fori_loop(0, NSB, sb_body, 0)
    pltpu.sync_copy(hist, counts_hbm.at[pl.ds(w * NB, NB)])


def _make_apply(next_shift):
    return functools.partial(
        pl.kernel,
        out_type=(
            jax.ShapeDtypeStruct((NP,), jnp.int32),
            jax.ShapeDtypeStruct((NP,), jnp.int32),
            jax.ShapeDtypeStruct((NW * NB,), jnp.int32),
        ),
        mesh=_MESH,
        scratch_types=[
            pltpu.VMEM((SB,), jnp.int32),
            pltpu.VMEM((SB,), jnp.int32),
            pltpu.VMEM((SB,), jnp.int32),
            pltpu.VMEM((SB,), jnp.int32),
            pltpu.VMEM((NB,), jnp.int32),
            pltpu.SemaphoreType.DMA,
        ],
        compiler_params=_SC_PARAMS,
    )(functools.partial(_apply_body, next_shift))


def _final_body(t_hbm, keys_hbm, inv1_hbm, inv2_hbm,
                s0, s1, s2, s3, s4, s5,
                codes_out, o0, o1, o2, o3, o4, o5,
                b0, b1, ivb, i2b, sxb, kb,
                g0, g1, g2, g3, g4, g5, sem, sem2):
    w = _wid()
    woff = w * CHUNK
    ones = jnp.ones((16,), jnp.int32)
    srcs = (s0, s1, s2, s3, s4, s5)
    outs = (o0, o1, o2, o3, o4, o5)
    bufs = (g0, g1, g2, g3, g4, g5)

    def sb_body(t, _):
        pltpu.sync_copy(t_hbm.at[pl.ds(woff + t * SB, SB)], b0)
        pltpu.sync_copy(t_hbm.at[pl.ds(NP + woff + t * SB, SB)], b1)

        def inv_body(q, _):
            s = pl.ds(q * 16, 16)
            ivb[s] = b0[s] + b1[s] - ones
            return 0

        lax.fori_loop(0, SB // 16, inv_body, 0)
        dk = pltpu.async_copy(keys_hbm.at[ivb], kb, sem)
        d2 = pltpu.async_copy(inv2_hbm.at[ivb], i2b, sem2)
        d2.wait()
        ds_ = pltpu.async_copy(inv1_hbm.at[i2b], sxb, sem2)
        dk.wait()
        pltpu.sync_copy(kb, codes_out.at[pl.ds(woff + t * SB, SB)])
        ds_.wait()

        def clamp_body(q, _):
            s = pl.ds(q * 16, 16)
            sxb[s] = jnp.minimum(sxb[s], N - 1)
            return 0

        lax.fori_loop(0, SB // 16, clamp_body, 0)
        descs = [pltpu.async_copy(s.at[sxb], b, sem)
                 for s, b in zip(srcs, bufs)]
        for d in descs:
            d.wait()
        for b, o in zip(bufs, outs):
            pltpu.sync_copy(b, o.at[pl.ds(woff + t * SB, SB)])
        return 0

    lax.fori_loop(0, NSB, sb_body, 0)


_final = functools.partial(
    pl.kernel,
    out_type=tuple(
        [jax.ShapeDtypeStruct((NP,), jnp.int32)]
        + [jax.ShapeDtypeStruct((NP,), jnp.float32) for _ in range(6)]
    ),
    mesh=_MESH,
    scratch_types=[pltpu.VMEM((SB,), jnp.int32) for _ in range(6)]
    + [pltpu.VMEM((SB,), jnp.float32) for _ in range(6)]
    + [pltpu.SemaphoreType.DMA, pltpu.SemaphoreType.DMA],
    compiler_params=_SC_PARAMS,
)(_final_body)


def _gath_body(sidx_hbm, s0, s1, s2, s3, s4, s5,
               o0, o1, o2, o3, o4, o5,
               iv0, iv1,
               a0, a1, a2, a3, a4, a5,
               c0, c1, c2, c3, c4, c5,
               sem0, sem1):
    w = _wid()
    woff = w * CHUNK
    srcs = (s0, s1, s2, s3, s4, s5)
    outs = (o0, o1, o2, o3, o4, o5)
    bufs = ((a0, a1, a2, a3, a4, a5), (c0, c1, c2, c3, c4, c5))
    ivs = (iv0, iv1)
    sems = (sem0, sem1)

    def _issue(t, b):
        pltpu.sync_copy(sidx_hbm.at[pl.ds(woff + t * SB, SB)], ivs[b])
        for s, buf in zip(srcs, bufs[b]):
            pltpu.async_copy(s.at[ivs[b]], buf, sems[b])

    _issue(0, 0)
    for t in range(NSB):
        b = t % 2
        if t + 1 < NSB:
            _issue(t + 1, 1 - b)
        for s, buf in zip(srcs, bufs[b]):
            pltpu.make_async_copy(s.at[ivs[b]], buf, sems[b]).wait()
        for buf, o in zip(bufs[b], outs):
            pltpu.sync_copy(buf, o.at[pl.ds(woff + t * SB, SB)])


_gath = functools.partial(
    pl.kernel,
    out_type=tuple(
        jax.ShapeDtypeStruct((NP,), jnp.float32) for _ in range(6)
    ),
    mesh=_MESH,
    scratch_types=[pltpu.VMEM((SB,), jnp.int32) for _ in range(2)]
    + [pltpu.VMEM((SB,), jnp.float32) for _ in range(12)]
    + [pltpu.SemaphoreType.DMA, pltpu.SemaphoreType.DMA],
    compiler_params=_SC_PARAMS,
)(_gath_body)

_hist0 = _make_hist(0)
_scat0 = _make_scat(0)
_scat1 = _make_scat(10)
_scat2 = _make_scat(20)
_apply0 = _make_apply(10)
_apply1 = _make_apply(20)


def kernel(pointcloud, color):
    pct = pointcloud.T
    pcp = jnp.pad(pct, ((0, 0), (0, NP - N)))
    codes = _morton_codes_padded(pcp).reshape(NP)
    colt = color.T

    c0 = _hist0(codes)
    t1 = _scat0(codes, c0)
    k1, inv1, c1 = _apply0(t1, codes)
    t2 = _scat1(k1, c1)
    k2, inv2, c2 = _apply1(t2, k1)
    t3 = _scat2(k2, c2)
    codes_s, ox, oy, oz, orr, og, ob = _final(
        t3, k2, inv1, inv2, pct[0], pct[1], pct[2], colt[0], colt[1], colt[2])
    pco = jnp.stack([ox, oy, oz], axis=1)
    colo = jnp.stack([orr, og, ob], axis=1)
    return (pco[:N], colo[:N], codes_s[:N].astype(jnp.uint32))


# speedup vs baseline: 7.5901x; 1.0205x over previous
import functools

import jax
import jax.numpy as jnp
from jax import lax
from jax.experimental import pallas as pl
from jax.experimental.pallas import tpu as pltpu
from jax.experimental.pallas import tpu_sc as plsc

N = 1_000_000
NP = 1_003_520
NW = 32
NC = 2
CHUNK = NP // NW
NB = 1024
SB = 6272
NSB = CHUNK // SB
ROWS = SB // 128
PAD_CODE = (1 << 30) - 1


C = 6272
G = NP // C


def _expand3(v):
    x = v
    x = (x | (x << 16)) & 0x30000FF
    x = (x | (x << 8)) & 0x300F00F
    x = (x | (x << 4)) & 0x30C30C3
    x = (x | (x << 2)) & 0x9249249
    return x


def _morton_body(pc_ref, codes_ref, mm_ref):
    ph = pl.program_id(0)
    g = pl.program_id(1)

    @pl.when(ph == 0)
    def _():
        blk = pc_ref[...]
        bmin = jnp.min(blk, axis=1, keepdims=True)
        bmax = jnp.max(blk, axis=1, keepdims=True)
        prev_min = jnp.where(g == 0, jnp.full_like(bmin, jnp.inf), mm_ref[:, 0:1])
        prev_max = jnp.where(g == 0, jnp.full_like(bmax, -jnp.inf), mm_ref[:, 1:2])
        mm_ref[:, 0:1] = jnp.minimum(prev_min, bmin)
        mm_ref[:, 1:2] = jnp.maximum(prev_max, bmax)

    @pl.when(ph == 1)
    def _():
        blk = pc_ref[...]
        bmin = mm_ref[:, 0:1]
        bmax = mm_ref[:, 1:2]
        scale = jnp.float32(1023) / (bmax - bmin + jnp.float32(1e-7))
        q = jnp.floor((blk - bmin) * scale).astype(jnp.int32)
        q = jnp.minimum(q, 1023)
        e = _expand3(q)
        code = (e[0:1, :] << 2) | (e[1:2, :] << 1) | e[2:3, :]
        pos = g * C + lax.broadcasted_iota(jnp.int32, (1, C), 1)
        codes_ref[...] = jnp.where(pos < N, code, PAD_CODE)


def _morton_codes_padded(pcp):
    return pl.pallas_call(
        _morton_body,
        grid=(2, G),
        in_specs=[pl.BlockSpec((3, C), lambda ph, g: (0, g))],
        out_specs=pl.BlockSpec((1, C), lambda ph, g: (0, g)),
        out_shape=jax.ShapeDtypeStruct((1, NP), jnp.int32),
        scratch_shapes=[pltpu.VMEM((3, 2), jnp.float32)],
    )(pcp)


_SC_PARAMS = pltpu.CompilerParams(
    needs_layout_passes=False, use_tc_tiling_on_sc=False
)
_MESH = plsc.VectorSubcoreMesh(core_axis_name="c", subcore_axis_name="s")


def _wid():
    return lax.axis_index("s") * NC + lax.axis_index("c")


def _hist_body(shift, keys_hbm, counts_hbm, hist, kb):
    w = _wid()
    woff = w * CHUNK
    zeros = jnp.zeros((16,), jnp.int32)
    ones = jnp.ones((16,), jnp.int32)

    def z(i, _):
        hist[pl.ds(i * 16, 16)] = zeros
        return 0

    lax.fori_loop(0, NB // 16, z, 0)

    def sb_body(t, _):
        pltpu.sync_copy(keys_hbm.at[pl.ds(woff + t * SB, SB)], kb)

        def v_body(q, _):
            k = kb[pl.ds(q * 16, 16)]
            d = (k >> shift) & (NB - 1)
            plsc.addupdate_scatter(hist, [d], ones)
            return 0

        lax.fori_loop(0, SB // 16, v_body, 0)
        return 0

    lax.fori_loop(0, NSB, sb_body, 0)
    pltpu.sync_copy(hist, counts_hbm.at[pl.ds(w * NB, NB)])


def _make_hist(shift):
    return functools.partial(
        pl.kernel,
        out_type=jax.ShapeDtypeStruct((NW * NB,), jnp.int32),
        mesh=_MESH,
        scratch_types=[
            pltpu.VMEM((NB,), jnp.int32),
            pltpu.VMEM((SB,), jnp.int32),
        ],
        compiler_params=_SC_PARAMS,
    )(functools.partial(_hist_body, shift))


WSLICE = NP // 16


def _scat_body(shift, keys_hbm, counts_hbm, t_out,
               counts_v, acc, base_r, kb, vb, db, zb, table, sem):
    w = _wid()
    woff = w * CHUNK
    zeros = jnp.zeros((16,), jnp.int32)
    ones = jnp.ones((16,), jnp.int32)

    pltpu.sync_copy(counts_hbm, counts_v)

    def z(i, _):
        acc[pl.ds(i * 16, 16)] = zeros
        base_r[pl.ds(i * 16, 16)] = zeros
        return 0

    lax.fori_loop(0, NB // 16, z, 0)

    def tot_w(wp, _):
        @pl.when(wp == w)
        def _():
            def cp(i, _):
                s = pl.ds(i * 16, 16)
                base_r[s] = acc[s]
                return 0

            lax.fori_loop(0, NB // 16, cp, 0)

        def tot_i(i, _):
            for u in range(4):
                s = pl.ds(i * 64 + u * 16, 16)
                acc[s] = acc[s] + counts_v[pl.ds(wp * NB + i * 64 + u * 16, 16)]
            return 0

        lax.fori_loop(0, NB // 64, tot_i, 0)
        return 0

    lax.fori_loop(0, NW, tot_w, 0)

    def scan_i(i, c):
        s = pl.ds(i * 16, 16)
        v = acc[s]
        cs = plsc.cumsum(v)
        base_r[s] = base_r[s] + (cs - v) + c
        return c + jnp.sum(v)

    lax.fori_loop(0, NB // 16, scan_i, jnp.int32(0))

    sid = lax.axis_index("s")

    def zz(i, _):
        zb[pl.ds(i * 16, 16)] = zeros
        return 0

    lax.fori_loop(0, SB // 16, zz, 0)

    def zt(i, _):
        pltpu.sync_copy(zb, table.at[pl.ds(sid * WSLICE + i * SB, SB)])
        return 0

    lax.fori_loop(0, WSLICE // SB, zt, 0)
    plsc.subcore_barrier()

    def sb_body(t, _):
        pltpu.sync_copy(keys_hbm.at[pl.ds(woff + t * SB, SB)], kb)

        def v_body(q, _):
            k = kb[pl.ds(q * 16, 16)]
            d = (k >> shift) & (NB - 1)
            g = plsc.load_gather(base_r, [d])
            cnt, _unused = plsc.scan_count(d)
            plsc.addupdate_scatter(base_r, [d], ones)
            db[pl.ds(q * 16, 16)] = g + cnt - ones
            vb[pl.ds(q * 16, 16)] = (woff + t * SB + q * 16 + 1) + lax.iota(
                jnp.int32, 16)
            return 0

        lax.fori_loop(0, SB // 16, v_body, 0)
        pltpu.async_copy(vb, table.at[db], sem).wait()
        return 0

    lax.fori_loop(0, NSB, sb_body, 0)
    plsc.subcore_barrier()

    cid = lax.axis_index("c")
    pltpu.sync_copy(table.at[pl.ds(sid * WSLICE, WSLICE)],
                    t_out.at[pl.ds(cid * NP + sid * WSLICE, WSLICE)])


def _make_scat(shift):
    return functools.partial(
        pl.kernel,
        out_type=jax.ShapeDtypeStruct((2 * NP,), jnp.int32),
        mesh=_MESH,
        scratch_types=[
            pltpu.VMEM((NW * NB,), jnp.int32),
            pltpu.VMEM((NB,), jnp.int32),
            pltpu.VMEM((NB,), jnp.int32),
            pltpu.VMEM((SB,), jnp.int32),
            pltpu.VMEM((SB,), jnp.int32),
            pltpu.VMEM((SB,), jnp.int32),
            pltpu.VMEM((SB,), jnp.int32),
            pltpu.VMEM_SHARED((NP,), jnp.int32),
            pltpu.SemaphoreType.DMA,
        ],
        compiler_params=_SC_PARAMS,
    )(functools.partial(_scat_body, shift))


def _apply_body(shift, t_hbm, keys_hbm, keys_out, inv_out, counts_hbm,
                b0, b1, ivb, kb, hist, sem):
    w = _wid()
    woff = w * CHUNK
    zeros = jnp.zeros((16,), jnp.int32)
    ones = jnp.ones((16,), jnp.int32)

    def z(i, _):
        hist[pl.ds(i * 16, 16)] = zeros
        return 0

    lax.fori_loop(0, NB // 16, z, 0)

    def sb_body(t, _):
        pltpu.sync_copy(t_hbm.at[pl.ds(woff + t * SB, SB)], b0)
        pltpu.sync_copy(t_hbm.at[pl.ds(NP + woff + t * SB, SB)], b1)

        def inv_body(q, _):
            for u in range(4):
                s = pl.ds(q * 64 + u * 16, 16)
                ivb[s] = b0[s] + b1[s] - ones
            return 0

        lax.fori_loop(0, SB // 64, inv_body, 0)
        pltpu.sync_copy(ivb, inv_out.at[pl.ds(woff + t * SB, SB)])
        pltpu.async_copy(keys_hbm.at[ivb], kb, sem).wait()

        def h_body(q, _):
            for u in range(4):
                k = kb[pl.ds(q * 64 + u * 16, 16)]
                d = (k >> shift) & (NB - 1)
                plsc.addupdate_scatter(hist, [d], ones)
            return 0

        lax.fori_loop(0, SB // 64, h_body, 0)
        pltpu.sync_copy(kb, keys_out.at[pl.ds(woff + t * SB, SB)])
        return 0

    lax.fori_loop(0, NSB, sb_body, 0)
    pltpu.sync_copy(hist, counts_hbm.at[pl.ds(w * NB, NB)])


def _make_apply(next_shift):
    return functools.partial(
        pl.kernel,
        out_type=(
            jax.ShapeDtypeStruct((NP,), jnp.int32),
            jax.ShapeDtypeStruct((NP,), jnp.int32),
            jax.ShapeDtypeStruct((NW * NB,), jnp.int32),
        ),
        mesh=_MESH,
        scratch_types=[
            pltpu.VMEM((SB,), jnp.int32),
            pltpu.VMEM((SB,), jnp.int32),
            pltpu.VMEM((SB,), jnp.int32),
            pltpu.VMEM((SB,), jnp.int32),
            pltpu.VMEM((NB,), jnp.int32),
            pltpu.SemaphoreType.DMA,
        ],
        compiler_params=_SC_PARAMS,
    )(functools.partial(_apply_body, next_shift))


def _final_body(t_hbm, keys_hbm, inv1_hbm, inv2_hbm,
                s0, s1, s2, s3, s4, s5,
                codes_out, o0, o1, o2, o3, o4, o5,
                b0, b1, ivb, i2b, sxb, kb,
                g0, g1, g2, g3, g4, g5, sem, sem2):
    w = _wid()
    woff = w * CHUNK
    ones = jnp.ones((16,), jnp.int32)
    srcs = (s0, s1, s2, s3, s4, s5)
    outs = (o0, o1, o2, o3, o4, o5)
    bufs = (g0, g1, g2, g3, g4, g5)

    def sb_body(t, _):
        pltpu.sync_copy(t_hbm.at[pl.ds(woff + t * SB, SB)], b0)
        pltpu.sync_copy(t_hbm.at[pl.ds(NP + woff + t * SB, SB)], b1)

        def inv_body(q, _):
            for u in range(4):
                s = pl.ds(q * 64 + u * 16, 16)
                ivb[s] = b0[s] + b1[s] - ones
            return 0

        lax.fori_loop(0, SB // 64, inv_body, 0)
        dk = pltpu.async_copy(keys_hbm.at[ivb], kb, sem)
        d2 = pltpu.async_copy(inv2_hbm.at[ivb], i2b, sem2)
        d2.wait()
        ds_ = pltpu.async_copy(inv1_hbm.at[i2b], sxb, sem2)
        dk.wait()
        pltpu.sync_copy(kb, codes_out.at[pl.ds(woff + t * SB, SB)])
        ds_.wait()

        def clamp_body(q, _):
            for u in range(4):
                s = pl.ds(q * 64 + u * 16, 16)
                sxb[s] = jnp.minimum(sxb[s], N - 1)
            return 0

        lax.fori_loop(0, SB // 64, clamp_body, 0)
        descs = [pltpu.async_copy(s.at[sxb], b, sem)
                 for s, b in zip(srcs, bufs)]
        for d in descs:
            d.wait()
        for b, o in zip(bufs, outs):
            pltpu.sync_copy(b, o.at[pl.ds(woff + t * SB, SB)])
        return 0

    lax.fori_loop(0, NSB, sb_body, 0)


_final = functools.partial(
    pl.kernel,
    out_type=tuple(
        [jax.ShapeDtypeStruct((NP,), jnp.int32)]
        + [jax.ShapeDtypeStruct((NP,), jnp.float32) for _ in range(6)]
    ),
    mesh=_MESH,
    scratch_types=[pltpu.VMEM((SB,), jnp.int32) for _ in range(6)]
    + [pltpu.VMEM((SB,), jnp.float32) for _ in range(6)]
    + [pltpu.SemaphoreType.DMA, pltpu.SemaphoreType.DMA],
    compiler_params=_SC_PARAMS,
)(_final_body)


def _gath_body(sidx_hbm, s0, s1, s2, s3, s4, s5,
               o0, o1, o2, o3, o4, o5,
               iv0, iv1,
               a0, a1, a2, a3, a4, a5,
               c0, c1, c2, c3, c4, c5,
               sem0, sem1):
    w = _wid()
    woff = w * CHUNK
    srcs = (s0, s1, s2, s3, s4, s5)
    outs = (o0, o1, o2, o3, o4, o5)
    bufs = ((a0, a1, a2, a3, a4, a5), (c0, c1, c2, c3, c4, c5))
    ivs = (iv0, iv1)
    sems = (sem0, sem1)

    def _issue(t, b):
        pltpu.sync_copy(sidx_hbm.at[pl.ds(woff + t * SB, SB)], ivs[b])
        for s, buf in zip(srcs, bufs[b]):
            pltpu.async_copy(s.at[ivs[b]], buf, sems[b])

    _issue(0, 0)
    for t in range(NSB):
        b = t % 2
        if t + 1 < NSB:
            _issue(t + 1, 1 - b)
        for s, buf in zip(srcs, bufs[b]):
            pltpu.make_async_copy(s.at[ivs[b]], buf, sems[b]).wait()
        for buf, o in zip(bufs[b], outs):
            pltpu.sync_copy(buf, o.at[pl.ds(woff + t * SB, SB)])


_gath = functools.partial(
    pl.kernel,
    out_type=tuple(
        jax.ShapeDtypeStruct((NP,), jnp.float32) for _ in range(6)
    ),
    mesh=_MESH,
    scratch_types=[pltpu.VMEM((SB,), jnp.int32) for _ in range(2)]
    + [pltpu.VMEM((SB,), jnp.float32) for _ in range(12)]
    + [pltpu.SemaphoreType.DMA, pltpu.SemaphoreType.DMA],
    compiler_params=_SC_PARAMS,
)(_gath_body)

_hist0 = _make_hist(0)
_scat0 = _make_scat(0)
_scat1 = _make_scat(10)
_scat2 = _make_scat(20)
_apply0 = _make_apply(10)
_apply1 = _make_apply(20)


def kernel(pointcloud, color):
    pct = pointcloud.T
    pcp = jnp.pad(pct, ((0, 0), (0, NP - N)))
    codes = _morton_codes_padded(pcp).reshape(NP)
    colt = color.T

    c0 = _hist0(codes)
    t1 = _scat0(codes, c0)
    k1, inv1, c1 = _apply0(t1, codes)
    t2 = _scat1(k1, c1)
    k2, inv2, c2 = _apply1(t2, k1)
    t3 = _scat2(k2, c2)
    codes_s, ox, oy, oz, orr, og, ob = _final(
        t3, k2, inv1, inv2, pct[0], pct[1], pct[2], colt[0], colt[1], colt[2])
    pco = jnp.stack([ox, oy, oz], axis=1)
    colo = jnp.stack([orr, og, ob], axis=1)
    return (pco[:N], colo[:N], codes_s[:N].astype(jnp.uint32))
